# baseline (speedup 1.0000x reference)
import functools

import jax
import jax.numpy as jnp
from jax import lax
from jax.experimental import pallas as pl
from jax.experimental.pallas import tpu as pltpu
from jax.experimental.pallas import tpu_sc as plsc

N = 10000
D = 256
E = 160000
NW = 32
NS = 16
C = 64
CHUNKS = 80
EW = C * CHUNKS
EPAD = NW * EW
L = 16

_mesh = plsc.VectorSubcoreMesh(core_axis_name="c", subcore_axis_name="s")


@functools.partial(
    pl.kernel,
    out_type=jax.ShapeDtypeStruct((EPAD,), jnp.float32),
    mesh=_mesh,
    compiler_params=pltpu.CompilerParams(use_tc_tiling_on_sc=False,
                                         needs_layout_passes=False),
    scratch_types=[
        pltpu.VMEM_SHARED((N, D), jnp.bfloat16),
        pltpu.VMEM((EW,), jnp.int32),
        pltpu.VMEM((EW,), jnp.int32),
        pltpu.VMEM((2 * C, D), jnp.bfloat16),
        pltpu.VMEM((2 * C, D), jnp.bfloat16),
        pltpu.VMEM((C,), jnp.float32),
        pltpu.VMEM((C,), jnp.float32),
        pltpu.SemaphoreType.DMA,
        pltpu.SemaphoreType.DMA,
        pltpu.SemaphoreType.DMA,
        pltpu.SemaphoreType.DMA,
    ],
)
def _decode(z_hbm, ei_hbm, out_hbm, z_sp, sidx, didx, rows0, rows1, oval0,
            oval1, sem0, sem1, osem0, osem1):
    cid = lax.axis_index("c")
    sid = lax.axis_index("s")
    wid = sid * 2 + cid
    base_e = wid * EW
    lanes = lax.iota(jnp.int32, L)

    rows_per_sub = N // NS
    pltpu.sync_copy(z_hbm.at[pl.ds(sid * rows_per_sub, rows_per_sub)],
                    z_sp.at[pl.ds(sid * rows_per_sub, rows_per_sub)])
    pltpu.sync_copy(ei_hbm.at[0, pl.ds(base_e, EW)], sidx)
    pltpu.sync_copy(ei_hbm.at[1, pl.ds(base_e, EW)], didx)
    plsc.subcore_barrier()

    def issue(j, buf, sem):
        pltpu.async_copy(z_sp.at[sidx.at[pl.ds(j * C, C)]],
                         buf.at[pl.ds(0, C)], sem)
        pltpu.async_copy(z_sp.at[didx.at[pl.ds(j * C, C)]],
                         buf.at[pl.ds(C, C)], sem)

    def drain(j, buf, sem):
        pltpu.make_async_copy(z_sp.at[sidx.at[pl.ds(j * C, C)]],
                              buf.at[pl.ds(0, C)], sem).wait()
        pltpu.make_async_copy(z_sp.at[didx.at[pl.ds(j * C, C)]],
                              buf.at[pl.ds(C, C)], sem).wait()

    def edge_dot(buf, e):
        p = buf[e, pl.ds(0, 2 * L)] * buf[e + C, pl.ds(0, 2 * L)]
        acc0, acc1 = plsc.unpack(p, format=plsc.PackFormat.INTERLEAVED)
        for q in range(1, D // (2 * L)):
            p = (buf[e, pl.ds(q * 2 * L, 2 * L)]
                 * buf[e + C, pl.ds(q * 2 * L, 2 * L)])
            a, b = plsc.unpack(p, format=plsc.PackFormat.INTERLEAVED)
            acc0 = acc0 + a
            acc1 = acc1 + b
        return jnp.sum(acc0 + acc1)

    def compute(j, buf, oval):
        for g in range(C // L):
            dots = jnp.zeros((L,), jnp.float32)
            for i in range(L):
                dots = jnp.where(lanes == i, edge_dot(buf, g * L + i), dots)
            oval[pl.ds(g * L, L)] = 1.0 / (1.0 + jnp.exp(-dots))

    def out_store(j, oval, osem):
        pltpu.async_copy(oval, out_hbm.at[pl.ds(base_e + j * C, C)], osem)

    def out_drain(j, oval, osem):
        pltpu.make_async_copy(
            oval, out_hbm.at[pl.ds(base_e + j * C, C)], osem).wait()

    issue(0, rows0, sem0)

    def pair_body(jj, _):
        j0 = 2 * jj
        issue(j0 + 1, rows1, sem1)
        drain(j0, rows0, sem0)
        @pl.when(jj > 0)
        def _():
            out_drain(j0 - 2, oval0, osem0)

        compute(j0, rows0, oval0)
        out_store(j0, oval0, osem0)
        @pl.when(jj + 1 < CHUNKS // 2)
        def _():
            issue(j0 + 2, rows0, sem0)

        drain(j0 + 1, rows1, sem1)
        @pl.when(jj > 0)
        def _():
            out_drain(j0 - 1, oval1, osem1)

        compute(j0 + 1, rows1, oval1)
        out_store(j0 + 1, oval1, osem1)
        return 0

    lax.fori_loop(0, CHUNKS // 2, pair_body, 0)
    out_drain(CHUNKS - 2, oval0, osem0)
    out_drain(CHUNKS - 1, oval1, osem1)


def kernel(z, edge_index):
    zb = z.astype(jnp.bfloat16)
    ei = edge_index.astype(jnp.int32)
    eip = jnp.pad(ei, ((0, 0), (0, EPAD - E)))
    return _decode(zb, eip)[:E]
